# baseline (speedup 1.0000x reference)
import jax
import jax.numpy as jnp
from jax import lax
from jax.experimental import pallas as pl
from jax.experimental.pallas import tpu as pltpu
from jax.experimental.pallas import tpu_sc as plsc

N_NODES = 10000
N_PAD = 10240
D = 128
DH = D // 2
N_EDGES = 320000
NS = 16
B = 128
BG = 256
EPT = 20480
CHUNKS = EPT // B
GCHUNKS = EPT // BG
E_PAD = EPT * NS
ROWS_PT = N_PAD // NS
NBUF = 2


def _bcast_lane(vec, lane):
    idx = jnp.full((16, 1), lane, jnp.int32)
    return lax.gather(
        vec, idx,
        lax.GatherDimensionNumbers(offset_dims=(), collapsed_slice_dims=(0,),
                                   start_index_map=(0,)),
        (1,), mode=lax.GatherScatterMode.PROMISE_IN_BOUNDS)


def _sc_body(xs, ss, ds_, ts, out,
             srcv, dstv, tbuf, rows, acc, gs0, gs1, ss0, ss1, ts0, ts1):
    cid = lax.axis_index("c")
    sid = lax.axis_index("s")
    gsem = (gs0, gs1)
    ssem = (ss0, ss1)
    tsem = (ts0, ts1)

    pltpu.sync_copy(ss.at[cid, pl.ds(sid * EPT, EPT)], srcv)
    cbase = sid * CHUNKS
    pltpu.sync_copy(ds_.at[cid, pl.ds(cbase, CHUNKS)], dstv)
    base = sid * ROWS_PT

    def one_half(h, _):
        x_hbm = xs.at[cid, h]

        def zrow(i, _):
            for j in range(DH // 16):
                rows[0, i, pl.ds(j * 16, 16)] = jnp.zeros((16,), jnp.float32)
            return 0
        lax.fori_loop(0, B, zrow, 0)
        for k in range(ROWS_PT // B):
            pltpu.sync_copy(rows.at[0, pl.ds(0, B)],
                            acc.at[pl.ds(base + k * B, B)])
        plsc.subcore_barrier()

        def gather_start(g, b):
            pltpu.async_copy(x_hbm.at[srcv.at[pl.ds(g * BG, BG)]],
                             rows.at[b], gsem[b])
            pltpu.async_copy(ts.at[cid, pl.ds(sid * EPT + g * BG, BG)],
                             tbuf.at[b], tsem[b])

        def gather_wait(b):
            pltpu.make_async_copy(x_hbm.at[pl.ds(0, BG)], rows.at[b],
                                  gsem[b]).wait()
            pltpu.make_async_copy(ts.at[cid, pl.ds(0, BG)], tbuf.at[b],
                                  tsem[b]).wait()

        def scatter_start(g, b):
            for q in range(BG // B):
                pltpu.async_copy(rows.at[b, pl.ds(q * B, B)],
                                 acc.at[dstv.at[g * (BG // B) + q]],
                                 ssem[b], add=True)

        def scatter_wait(b):
            for _q in range(BG // B):
                pltpu.make_async_copy(rows.at[b, pl.ds(0, B)],
                                      acc.at[pl.ds(0, B)], ssem[b]).wait()

        def scale(g, b):
            def mgroup(m, _):
                tvec = tbuf[b, pl.ds(m * 16, 16)]
                for lane in range(16):
                    tb = _bcast_lane(tvec, lane)
                    r = m * 16 + lane
                    for j in range(DH // 16):
                        sl = pl.ds(j * 16, 16)
                        rows[b, r, sl] = rows[b, r, sl] * tb
                return 0
            lax.fori_loop(0, BG // 16, mgroup, 0)

        gather_start(0, 0)

        def outer(g0, _):
            for b in range(NBUF):
                g = g0 * NBUF + b

                @pl.when(g > 0)
                def _():
                    scatter_wait(1 - b)

                @pl.when(g < GCHUNKS - 1)
                def _():
                    gather_start(g + 1, 1 - b)

                gather_wait(b)
                scale(g, b)
                scatter_start(g, b)
            return 0
        lax.fori_loop(0, GCHUNKS // NBUF, outer, 0)
        scatter_wait(1)

        plsc.subcore_barrier()
        pltpu.sync_copy(acc.at[pl.ds(base, ROWS_PT)],
                        out.at[cid, h, pl.ds(base, ROWS_PT)])
        return 0

    lax.fori_loop(0, 2, one_half, 0)


@jax.jit
def _stack_aggregate(xs, ss, ds_, ts):
    mesh = plsc.VectorSubcoreMesh(core_axis_name="c", subcore_axis_name="s")
    f = pl.kernel(
        _sc_body,
        out_type=jax.ShapeDtypeStruct((2, 2, N_PAD, DH), jnp.float32),
        mesh=mesh,
        compiler_params=pltpu.CompilerParams(use_tc_tiling_on_sc=False),
        scratch_types=[
            pltpu.VMEM((EPT,), jnp.int32),
            pltpu.VMEM((CHUNKS, B), jnp.int32),
            pltpu.VMEM((NBUF, BG), jnp.float32),
            pltpu.VMEM((NBUF, BG, DH), jnp.float32),
            pltpu.VMEM_SHARED((N_PAD, DH), jnp.float32),
        ] + [pltpu.SemaphoreType.DMA] * (3 * NBUF),
    )
    return f(xs, ss, ds_, ts)


def _prep(edge_index, t):
    src = edge_index[0].astype(jnp.int32)
    dst = edge_index[1].astype(jnp.int32)
    tt = t.reshape(-1).astype(jnp.float32)
    pad = E_PAD - N_EDGES
    src = jnp.concatenate([src, jnp.zeros((pad,), jnp.int32)])
    dst = jnp.concatenate([dst, jnp.zeros((pad,), jnp.int32)]).reshape(-1, B)
    tt = jnp.concatenate([tt, jnp.zeros((pad,), jnp.float32)])
    return src, dst, tt


def kernel(x0, x1, edge_index0, edge_index1, t0, t1):
    s0, d0, tt0 = _prep(edge_index0, t0)
    s1, d1, tt1 = _prep(edge_index1, t1)
    xs = jnp.stack([
        jnp.stack([x0[:, :DH], x0[:, DH:]]),
        jnp.stack([x1[:, :DH], x1[:, DH:]]),
    ])
    out = _stack_aggregate(xs,
                           jnp.stack([s0, s1]),
                           jnp.stack([d0, d1]),
                           jnp.stack([tt0, tt1]))
    hm0 = jnp.concatenate([out[0, 0, :N_NODES], out[0, 1, :N_NODES]], axis=1)
    hm1 = jnp.concatenate([out[1, 0, :N_NODES], out[1, 1, :N_NODES]], axis=1)
    return jnp.stack([hm0, hm1], axis=1)


# speedup vs baseline: 1.2074x; 1.2074x over previous
import jax
import jax.numpy as jnp
from jax import lax
from jax.experimental import pallas as pl
from jax.experimental.pallas import tpu as pltpu
from jax.experimental.pallas import tpu_sc as plsc

N_NODES = 10000
N_PAD = 10240
D = 128
DH = D // 2
N_EDGES = 320000
NS = 16
B = 128
EPT = 20480
CHUNKS = EPT // B
E_PAD = EPT * NS
ROWS_PT = N_PAD // NS


def _bcast_lane(vec, lane):
    idx = jnp.full((16, 1), lane, jnp.int32)
    return lax.gather(
        vec, idx,
        lax.GatherDimensionNumbers(offset_dims=(), collapsed_slice_dims=(0,),
                                   start_index_map=(0,)),
        (1,), mode=lax.GatherScatterMode.PROMISE_IN_BOUNDS)


def _sc_body(xs, ss, ds_, ts, out,
             srcv, dstv, tv, rows, zbuf, acc, gs0, gs1, ss0, ss1):
    cid = lax.axis_index("c")
    sid = lax.axis_index("s")
    gsem = (gs0, gs1)
    ssem = (ss0, ss1)

    cbase = sid * CHUNKS
    pltpu.sync_copy(ss.at[cid, pl.ds(cbase, CHUNKS)], srcv)
    pltpu.sync_copy(ds_.at[cid, pl.ds(cbase, CHUNKS)], dstv)
    pltpu.sync_copy(ts.at[cid, pl.ds(cbase, CHUNKS)], tv)
    base = sid * ROWS_PT

    def zrow(i, _):
        for j in range(DH // 16):
            zbuf[i, pl.ds(j * 16, 16)] = jnp.zeros((16,), jnp.float32)
        return 0
    lax.fori_loop(0, B, zrow, 0)

    def one_half(h, _):
        x_hbm = xs.at[cid, h]

        for k in range(ROWS_PT // B):
            pltpu.sync_copy(zbuf, acc.at[pl.ds(base + k * B, B)])
        plsc.subcore_barrier()

        def gather_start(c, b):
            pltpu.async_copy(x_hbm.at[srcv.at[c]], rows.at[b], gsem[b])

        def gather_wait(b):
            pltpu.make_async_copy(x_hbm.at[pl.ds(0, B)], rows.at[b],
                                  gsem[b]).wait()

        def scatter_start(c, b):
            pltpu.async_copy(rows.at[b], acc.at[dstv.at[c]], ssem[b],
                             add=True)

        def scatter_wait(b):
            pltpu.make_async_copy(rows.at[b], acc.at[pl.ds(0, B)],
                                  ssem[b]).wait()

        def scale(c, b):
            for g in range(B // 16):
                tvec = tv[c, pl.ds(g * 16, 16)]
                for lane in range(16):
                    tb = _bcast_lane(tvec, lane)
                    r = g * 16 + lane
                    for j in range(DH // 16):
                        sl = pl.ds(j * 16, 16)
                        rows[b, r, sl] = rows[b, r, sl] * tb

        gather_start(0, 0)

        def outer(c0, _):
            for b in range(2):
                c = c0 * 2 + b

                @pl.when(c > 0)
                def _():
                    scatter_wait(1 - b)

                @pl.when(c < CHUNKS - 1)
                def _():
                    gather_start(c + 1, 1 - b)

                gather_wait(b)
                scale(c, b)
                scatter_start(c, b)
            return 0
        lax.fori_loop(0, CHUNKS // 2, outer, 0)
        scatter_wait(1)

        plsc.subcore_barrier()
        pltpu.sync_copy(acc.at[pl.ds(base, ROWS_PT)],
                        out.at[cid, h, pl.ds(base, ROWS_PT)])
        return 0

    lax.fori_loop(0, 2, one_half, 0)


@jax.jit
def _stack_aggregate(xs, ss, ds_, ts):
    mesh = plsc.VectorSubcoreMesh(core_axis_name="c", subcore_axis_name="s")
    f = pl.kernel(
        _sc_body,
        out_type=jax.ShapeDtypeStruct((2, 2, N_PAD, DH), jnp.float32),
        mesh=mesh,
        compiler_params=pltpu.CompilerParams(use_tc_tiling_on_sc=False),
        scratch_types=[
            pltpu.VMEM((CHUNKS, B), jnp.int32),
            pltpu.VMEM((CHUNKS, B), jnp.int32),
            pltpu.VMEM((CHUNKS, B), jnp.float32),
            pltpu.VMEM((2, B, DH), jnp.float32),
            pltpu.VMEM((B, DH), jnp.float32),
            pltpu.VMEM_SHARED((N_PAD, DH), jnp.float32),
            pltpu.SemaphoreType.DMA,
            pltpu.SemaphoreType.DMA,
            pltpu.SemaphoreType.DMA,
            pltpu.SemaphoreType.DMA,
        ],
    )
    return f(xs, ss, ds_, ts)


def _prep(edge_index, t):
    src = edge_index[0].astype(jnp.int32)
    dst = edge_index[1].astype(jnp.int32)
    tt = t.reshape(-1).astype(jnp.float32)
    pad = E_PAD - N_EDGES
    src = jnp.concatenate([src, jnp.zeros((pad,), jnp.int32)]).reshape(-1, B)
    dst = jnp.concatenate([dst, jnp.zeros((pad,), jnp.int32)]).reshape(-1, B)
    tt = jnp.concatenate([tt, jnp.zeros((pad,), jnp.float32)]).reshape(-1, B)
    return src, dst, tt


def kernel(x0, x1, edge_index0, edge_index1, t0, t1):
    s0, d0, tt0 = _prep(edge_index0, t0)
    s1, d1, tt1 = _prep(edge_index1, t1)
    xs = jnp.stack([
        jnp.stack([x0[:, :DH], x0[:, DH:]]),
        jnp.stack([x1[:, :DH], x1[:, DH:]]),
    ])
    out = _stack_aggregate(xs,
                           jnp.stack([s0, s1]),
                           jnp.stack([d0, d1]),
                           jnp.stack([tt0, tt1]))
    hm0 = jnp.concatenate([out[0, 0, :N_NODES], out[0, 1, :N_NODES]], axis=1)
    hm1 = jnp.concatenate([out[1, 0, :N_NODES], out[1, 1, :N_NODES]], axis=1)
    return jnp.stack([hm0, hm1], axis=1)
